# baseline (speedup 1.0000x reference)
import functools

import jax
import jax.numpy as jnp
from jax import lax
from jax.experimental import pallas as pl
from jax.experimental.pallas import tpu as pltpu
from jax.experimental.pallas import tpu_sc as plsc

_VOCAB = 1000
_VVOCAB = 128
_EMB = 64
_HID = 512
_B = 1024
_T = 20
_TP = 24
_DB = 16
_CHUNK = 128
_EMBP = 128
_WAVE = 5


def _sc_gather_rows(table, idx):
    info = plsc.get_sparse_core_info()
    nc, ns = info.num_cores, info.num_subcores
    nw = nc * ns
    k = idx.shape[1]
    mesh = plsc.VectorSubcoreMesh(core_axis_name="c", subcore_axis_name="s")

    @functools.partial(
        pl.kernel,
        mesh=mesh,
        out_type=jax.ShapeDtypeStruct((nw, k, _CHUNK, _EMBP), jnp.float32),
        scratch_types=[
            pltpu.VMEM((k, _CHUNK), jnp.int32),
            pltpu.VMEM((_WAVE, _CHUNK, _EMBP), jnp.float32),
            pltpu.SemaphoreType.DMA,
        ],
    )
    def run(table_hbm, idx_hbm, out_hbm, idx_v, rows_v, sem):
        wid = lax.axis_index("s") * nc + lax.axis_index("c")
        pltpu.sync_copy(idx_hbm.at[wid], idx_v)
        for w in range(k // _WAVE):
            cps = [
                pltpu.async_copy(
                    table_hbm.at[idx_v.at[w * _WAVE + j]], rows_v.at[j], sem)
                for j in range(_WAVE)
            ]
            for cp in cps:
                cp.wait()
            pltpu.sync_copy(rows_v, out_hbm.at[wid].at[pl.ds(w * _WAVE, _WAVE)])

    return run(table, idx)


_NSPLIT = 4


def _sigmoid(x):
    return jnp.tanh(x * 0.5) * 0.5 + 0.5


def _lstm_body(g_ref, wih_ref, whh_ref, bg_ref, hall_ref, ht_ref, ct_ref,
               h_sc, c_sc, hacc):
    t = pl.program_id(0)
    tm = t % 8

    @pl.when(t == 0)
    def _():
        h_sc[...] = jnp.zeros_like(h_sc)
        c_sc[...] = jnp.zeros_like(c_sc)

    bs = _B // _NSPLIT
    for s in range(_NSPLIT):
        r0, r1 = s * bs, (s + 1) * bs
        x = (g_ref[0, 0, r0:r1, :] + g_ref[1, 0, r0:r1, :]).astype(jnp.bfloat16)
        h = h_sc[r0:r1, :]
        c = c_sc[r0:r1, :]
        gates = (jnp.dot(x, wih_ref[...], preferred_element_type=jnp.float32)
                 + jnp.dot(h, whh_ref[...], preferred_element_type=jnp.float32)
                 + bg_ref[...])
        gif = _sigmoid(gates[:, :2 * _HID])
        gg = jnp.tanh(gates[:, 2 * _HID:3 * _HID])
        go = _sigmoid(gates[:, 3 * _HID:])
        c2 = gif[:, _HID:] * c + gif[:, :_HID] * gg
        h2 = go * jnp.tanh(c2)
        h2b = h2.astype(jnp.bfloat16)
        h_sc[r0:r1, :] = h2b
        c_sc[r0:r1, :] = c2
        for q in range(8):
            @pl.when(tm == q)
            def _(q=q, h2b=h2b, r0=r0, r1=r1):
                hacc[q, r0:r1, :] = h2b

        @pl.when(t == _T - 1)
        def _(h2=h2, c2=c2, r0=r0, r1=r1):
            ht_ref[0, r0:r1, :] = h2
            ct_ref[0, r0:r1, :] = c2

    @pl.when((tm == 7) | (t == _T - 1))
    def _():
        hall_ref[...] = jnp.swapaxes(hacc[...], 0, 1)


def _lstm_call(g, wihT, whhT, bg):
    return pl.pallas_call(
        _lstm_body,
        grid=(_T,),
        in_specs=[
            pl.BlockSpec((2, 1, _B, _EMBP), lambda t: (0, t, 0, 0)),
            pl.BlockSpec((_EMBP, 4 * _HID), lambda t: (0, 0)),
            pl.BlockSpec((_HID, 4 * _HID), lambda t: (0, 0)),
            pl.BlockSpec((1, 4 * _HID), lambda t: (0, 0)),
        ],
        out_specs=[
            pl.BlockSpec((_B, 8, _HID), lambda t: (0, t // 8, 0)),
            pl.BlockSpec((1, _B, _HID), lambda t: (0, 0, 0)),
            pl.BlockSpec((1, _B, _HID), lambda t: (0, 0, 0)),
        ],
        out_shape=[
            jax.ShapeDtypeStruct((_B, _TP, _HID), jnp.bfloat16),
            jax.ShapeDtypeStruct((1, _B, _HID), jnp.float32),
            jax.ShapeDtypeStruct((1, _B, _HID), jnp.float32),
        ],
        scratch_shapes=[
            pltpu.VMEM((_B, _HID), jnp.bfloat16),
            pltpu.VMEM((_B, _HID), jnp.float32),
            pltpu.VMEM((8, _B, _HID), jnp.bfloat16),
        ],
        compiler_params=pltpu.CompilerParams(
            dimension_semantics=("arbitrary",),
            vmem_limit_bytes=100 * 1024 * 1024,
        ),
    )(g, wihT, whhT, bg)


def _dec_body(h_ref, wd_ref, bd_ref, wv_ref, bv_ref, out_ref, outv_ref):
    rows = h_ref[...]
    logits = jnp.dot(rows, wd_ref[...], preferred_element_type=jnp.float32) + bd_ref[...]
    sh = logits - jnp.max(logits, axis=-1, keepdims=True)
    sm = sh - jnp.log(jnp.sum(jnp.exp(sh), axis=-1, keepdims=True))
    vlog = jnp.dot(rows, wv_ref[...], preferred_element_type=jnp.float32) + bv_ref[...]
    vsh = vlog - jnp.max(vlog, axis=-1, keepdims=True)
    vm = vsh - jnp.log(jnp.sum(jnp.exp(vsh), axis=-1, keepdims=True))
    for bb in range(_DB):
        out_ref[bb] = sm[bb * _TP:bb * _TP + _T, :]
        outv_ref[bb] = vm[bb * _TP:bb * _TP + _T, :]


def _dec_call(hrows, wdT, bd, wvT, bv):
    nb = _B // _DB
    return pl.pallas_call(
        _dec_body,
        grid=(nb,),
        in_specs=[
            pl.BlockSpec((_DB * _TP, _HID), lambda i: (i, 0)),
            pl.BlockSpec((_HID, _VOCAB), lambda i: (0, 0)),
            pl.BlockSpec((1, _VOCAB), lambda i: (0, 0)),
            pl.BlockSpec((_HID, _VVOCAB), lambda i: (0, 0)),
            pl.BlockSpec((1, _VVOCAB), lambda i: (0, 0)),
        ],
        out_specs=[
            pl.BlockSpec((_DB, _T, _VOCAB), lambda i: (i, 0, 0)),
            pl.BlockSpec((_DB, _T, _VVOCAB), lambda i: (i, 0, 0)),
        ],
        out_shape=[
            jax.ShapeDtypeStruct((_B, _T, _VOCAB), jnp.float32),
            jax.ShapeDtypeStruct((_B, _T, _VVOCAB), jnp.float32),
        ],
        compiler_params=pltpu.CompilerParams(
            dimension_semantics=("arbitrary",),
            vmem_limit_bytes=100 * 1024 * 1024,
        ),
    )(hrows, wdT, bd, wvT, bv)


def kernel(x1, x2, encoder, encoder_vel, W_ih, W_hh, b_ih, b_hh, W_dec, b_dec,
           W_dec_vel, b_dec_vel):
    info = plsc.get_sparse_core_info()
    nw = info.num_cores * info.num_subcores
    idx = jnp.concatenate([
        jnp.swapaxes(x1, 0, 1).reshape(-1),
        jnp.swapaxes(x2, 0, 1).reshape(-1),
    ]).astype(jnp.int32)
    idx3 = idx.reshape(nw, -1, _CHUNK)
    table_p = jnp.pad(encoder, ((0, 0), (0, _EMBP - _EMB)))
    rows = _sc_gather_rows(table_p, idx3)
    g = rows.reshape(2, _T, _B, _EMBP)

    bg = (b_ih + b_hh).reshape(1, 4 * _HID)
    bf16 = jnp.bfloat16
    wihT_p = jnp.pad(W_ih.T, ((0, _EMBP - _EMB), (0, 0))).astype(bf16)
    hall, ht, ct = _lstm_call(g, wihT_p, W_hh.T.astype(bf16), bg)

    hrows = hall.reshape(_B * _TP, _HID)

    out, outv = _dec_call(
        hrows, W_dec.T.astype(bf16), b_dec.reshape(1, _VOCAB),
        W_dec_vel.T.astype(bf16), b_dec_vel.reshape(1, _VVOCAB))
    return (out, outv, (ht, ct))


# speedup vs baseline: 1.0235x; 1.0235x over previous
import functools

import jax
import jax.numpy as jnp
from jax import lax
from jax.experimental import pallas as pl
from jax.experimental.pallas import tpu as pltpu
from jax.experimental.pallas import tpu_sc as plsc

_VOCAB = 1000
_VVOCAB = 128
_EMB = 64
_HID = 512
_B = 1024
_T = 20
_TP = 24
_DB = 16
_CHUNK = 128
_EMBP = 128
_WAVE = 5


def _sc_gather_rows(table, idx):
    info = plsc.get_sparse_core_info()
    nc, ns = info.num_cores, info.num_subcores
    nw = nc * ns
    k = idx.shape[1]
    mesh = plsc.VectorSubcoreMesh(core_axis_name="c", subcore_axis_name="s")

    @functools.partial(
        pl.kernel,
        mesh=mesh,
        out_type=jax.ShapeDtypeStruct((nw, k, _CHUNK, _EMBP), jnp.float32),
        scratch_types=[
            pltpu.VMEM((k, _CHUNK), jnp.int32),
            pltpu.VMEM((_WAVE, _CHUNK, _EMBP), jnp.float32),
            pltpu.SemaphoreType.DMA,
        ],
    )
    def run(table_hbm, idx_hbm, out_hbm, idx_v, rows_v, sem):
        wid = lax.axis_index("s") * nc + lax.axis_index("c")
        pltpu.sync_copy(idx_hbm.at[wid], idx_v)
        for w in range(k // _WAVE):
            cps = [
                pltpu.async_copy(
                    table_hbm.at[idx_v.at[w * _WAVE + j]], rows_v.at[j], sem)
                for j in range(_WAVE)
            ]
            for cp in cps:
                cp.wait()
            pltpu.sync_copy(rows_v, out_hbm.at[wid].at[pl.ds(w * _WAVE, _WAVE)])

    return run(table, idx)


_NSPLIT = 2


def _sigmoid(x):
    return jnp.tanh(x * 0.5) * 0.5 + 0.5


def _lstm_body(g_ref, wih_ref, whh_ref, bg_ref, hall_ref, ht_ref, ct_ref,
               h_sc, c_sc, hacc):
    t = pl.program_id(0)
    tm = t % 8

    @pl.when(t == 0)
    def _():
        h_sc[...] = jnp.zeros_like(h_sc)
        c_sc[...] = jnp.zeros_like(c_sc)

    bs = _B // _NSPLIT
    for s in range(_NSPLIT):
        r0, r1 = s * bs, (s + 1) * bs
        x = (g_ref[0, 0, r0:r1, :] + g_ref[1, 0, r0:r1, :]).astype(jnp.bfloat16)
        h = h_sc[r0:r1, :]
        c = c_sc[r0:r1, :]
        nt = (((1,), (1,)), ((), ()))
        gates = (lax.dot_general(x, wih_ref[...], nt,
                                 preferred_element_type=jnp.float32)
                 + lax.dot_general(h, whh_ref[...], nt,
                                   preferred_element_type=jnp.float32)
                 + bg_ref[...])
        gif = _sigmoid(gates[:, :2 * _HID])
        gg = jnp.tanh(gates[:, 2 * _HID:3 * _HID])
        go = _sigmoid(gates[:, 3 * _HID:])
        c2 = gif[:, _HID:] * c + gif[:, :_HID] * gg
        h2 = go * jnp.tanh(c2)
        h2b = h2.astype(jnp.bfloat16)
        h_sc[r0:r1, :] = h2b
        c_sc[r0:r1, :] = c2
        for q in range(8):
            @pl.when(tm == q)
            def _(q=q, h2b=h2b, r0=r0, r1=r1):
                hacc[q, r0:r1, :] = h2b

        @pl.when(t == _T - 1)
        def _(h2=h2, c2=c2, r0=r0, r1=r1):
            ht_ref[0, r0:r1, :] = h2
            ct_ref[0, r0:r1, :] = c2

    @pl.when((tm == 7) | (t == _T - 1))
    def _():
        hall_ref[...] = jnp.swapaxes(hacc[...], 0, 1)


def _lstm_call(g, wihT, whhT, bg):
    return pl.pallas_call(
        _lstm_body,
        grid=(_T,),
        in_specs=[
            pl.BlockSpec((2, 1, _B, _EMBP), lambda t: (0, t, 0, 0)),
            pl.BlockSpec((4 * _HID, _EMBP), lambda t: (0, 0)),
            pl.BlockSpec((4 * _HID, _HID), lambda t: (0, 0)),
            pl.BlockSpec((1, 4 * _HID), lambda t: (0, 0)),
        ],
        out_specs=[
            pl.BlockSpec((_B, 8, _HID), lambda t: (0, t // 8, 0)),
            pl.BlockSpec((1, _B, _HID), lambda t: (0, 0, 0)),
            pl.BlockSpec((1, _B, _HID), lambda t: (0, 0, 0)),
        ],
        out_shape=[
            jax.ShapeDtypeStruct((_B, _TP, _HID), jnp.bfloat16),
            jax.ShapeDtypeStruct((1, _B, _HID), jnp.float32),
            jax.ShapeDtypeStruct((1, _B, _HID), jnp.float32),
        ],
        scratch_shapes=[
            pltpu.VMEM((_B, _HID), jnp.bfloat16),
            pltpu.VMEM((_B, _HID), jnp.float32),
            pltpu.VMEM((8, _B, _HID), jnp.bfloat16),
        ],
        compiler_params=pltpu.CompilerParams(
            dimension_semantics=("arbitrary",),
            vmem_limit_bytes=100 * 1024 * 1024,
        ),
    )(g, wihT, whhT, bg)


def _dec_body(h_ref, wd_ref, bd_ref, wv_ref, bv_ref, out_ref, outv_ref):
    rows = h_ref[...]
    nt = (((1,), (1,)), ((), ()))
    logits = lax.dot_general(rows, wd_ref[...], nt,
                             preferred_element_type=jnp.float32) + bd_ref[...]
    sh = logits - jnp.max(logits, axis=-1, keepdims=True)
    sm = sh - jnp.log(jnp.sum(jnp.exp(sh), axis=-1, keepdims=True))
    vlog = lax.dot_general(rows, wv_ref[...], nt,
                           preferred_element_type=jnp.float32) + bv_ref[...]
    vsh = vlog - jnp.max(vlog, axis=-1, keepdims=True)
    vm = vsh - jnp.log(jnp.sum(jnp.exp(vsh), axis=-1, keepdims=True))
    for bb in range(_DB):
        out_ref[bb] = sm[bb * _TP:bb * _TP + _T, :]
        outv_ref[bb] = vm[bb * _TP:bb * _TP + _T, :]


def _dec_call(hrows, wdT, bd, wvT, bv):
    nb = _B // _DB
    return pl.pallas_call(
        _dec_body,
        grid=(nb,),
        in_specs=[
            pl.BlockSpec((_DB * _TP, _HID), lambda i: (i, 0)),
            pl.BlockSpec((_VOCAB, _HID), lambda i: (0, 0)),
            pl.BlockSpec((1, _VOCAB), lambda i: (0, 0)),
            pl.BlockSpec((_VVOCAB, _HID), lambda i: (0, 0)),
            pl.BlockSpec((1, _VVOCAB), lambda i: (0, 0)),
        ],
        out_specs=[
            pl.BlockSpec((_DB, _T, _VOCAB), lambda i: (i, 0, 0)),
            pl.BlockSpec((_DB, _T, _VVOCAB), lambda i: (i, 0, 0)),
        ],
        out_shape=[
            jax.ShapeDtypeStruct((_B, _T, _VOCAB), jnp.float32),
            jax.ShapeDtypeStruct((_B, _T, _VVOCAB), jnp.float32),
        ],
        compiler_params=pltpu.CompilerParams(
            dimension_semantics=("arbitrary",),
            vmem_limit_bytes=100 * 1024 * 1024,
        ),
    )(hrows, wdT, bd, wvT, bv)


def kernel(x1, x2, encoder, encoder_vel, W_ih, W_hh, b_ih, b_hh, W_dec, b_dec,
           W_dec_vel, b_dec_vel):
    info = plsc.get_sparse_core_info()
    nw = info.num_cores * info.num_subcores
    idx = jnp.concatenate([
        jnp.swapaxes(x1, 0, 1).reshape(-1),
        jnp.swapaxes(x2, 0, 1).reshape(-1),
    ]).astype(jnp.int32)
    idx3 = idx.reshape(nw, -1, _CHUNK)
    table_p = jnp.pad(encoder, ((0, 0), (0, _EMBP - _EMB)))
    rows = _sc_gather_rows(table_p, idx3)
    g = rows.reshape(2, _T, _B, _EMBP)

    bg = (b_ih + b_hh).reshape(1, 4 * _HID)
    bf16 = jnp.bfloat16
    wih_p = jnp.pad(W_ih, ((0, 0), (0, _EMBP - _EMB))).astype(bf16)
    hall, ht, ct = _lstm_call(g, wih_p, W_hh.astype(bf16), bg)

    hrows = hall.reshape(_B * _TP, _HID)

    out, outv = _dec_call(
        hrows, W_dec.astype(bf16), b_dec.reshape(1, _VOCAB),
        W_dec_vel.astype(bf16), b_dec_vel.reshape(1, _VVOCAB))
    return (out, outv, (ht, ct))


# speedup vs baseline: 1.5525x; 1.5169x over previous
import functools

import jax
import jax.numpy as jnp
from jax import lax
from jax.experimental import pallas as pl
from jax.experimental.pallas import tpu as pltpu
from jax.experimental.pallas import tpu_sc as plsc

_VOCAB = 1000
_VVOCAB = 128
_EMB = 64
_HID = 512
_B = 1024
_T = 20
_TP = 24
_DB = 16
_CHUNK = 128
_EMBP = 128
_WAVE = 5


def _sc_gather_rows(table, idx):
    info = plsc.get_sparse_core_info()
    nc, ns = info.num_cores, info.num_subcores
    nw = nc * ns
    k = idx.shape[1]
    mesh = plsc.VectorSubcoreMesh(core_axis_name="c", subcore_axis_name="s")

    @functools.partial(
        pl.kernel,
        mesh=mesh,
        out_type=jax.ShapeDtypeStruct((nw, k, _CHUNK, _EMBP), jnp.float32),
        scratch_types=[
            pltpu.VMEM((k, _CHUNK), jnp.int32),
            pltpu.VMEM((_WAVE, _CHUNK, _EMBP), jnp.float32),
            pltpu.SemaphoreType.DMA,
        ],
    )
    def run(table_hbm, idx_hbm, out_hbm, idx_v, rows_v, sem):
        wid = lax.axis_index("s") * nc + lax.axis_index("c")
        pltpu.sync_copy(idx_hbm.at[wid], idx_v)
        for w in range(k // _WAVE):
            cps = [
                pltpu.async_copy(
                    table_hbm.at[idx_v.at[w * _WAVE + j]], rows_v.at[j], sem)
                for j in range(_WAVE)
            ]
            for cp in cps:
                cp.wait()
            pltpu.sync_copy(rows_v, out_hbm.at[wid].at[pl.ds(w * _WAVE, _WAVE)])

    return run(table, idx)


_NSPLIT = 2


def _sigmoid(x):
    return jnp.tanh(x * 0.5) * 0.5 + 0.5


def _lstm_body(g_ref, wih_ref, whh_ref, bg_ref, hall_ref, ht_ref, ct_ref,
               h_sc, c_sc):
    t = pl.program_id(0)

    @pl.when(t == 0)
    def _():
        h_sc[...] = jnp.zeros_like(h_sc)
        c_sc[...] = jnp.zeros_like(c_sc)

    bs = _B // _NSPLIT
    for s in range(_NSPLIT):
        r0, r1 = s * bs, (s + 1) * bs
        x = (g_ref[0, 0, r0:r1, :] + g_ref[1, 0, r0:r1, :]).astype(jnp.bfloat16)
        h = h_sc[r0:r1, :]
        c = c_sc[r0:r1, :]
        nt = (((1,), (1,)), ((), ()))
        gates = (lax.dot_general(x, wih_ref[...], nt,
                                 preferred_element_type=jnp.float32)
                 + lax.dot_general(h, whh_ref[...], nt,
                                   preferred_element_type=jnp.float32)
                 + bg_ref[...])
        gif = _sigmoid(gates[:, :2 * _HID])
        gg = jnp.tanh(gates[:, 2 * _HID:3 * _HID])
        go = _sigmoid(gates[:, 3 * _HID:])
        c2 = gif[:, _HID:] * c + gif[:, :_HID] * gg
        h2 = go * jnp.tanh(c2)
        h2b = h2.astype(jnp.bfloat16)
        h_sc[r0:r1, :] = h2b
        c_sc[r0:r1, :] = c2
        hall_ref[0, r0:r1, :] = h2b

        @pl.when(t == _T - 1)
        def _(h2=h2, c2=c2, r0=r0, r1=r1):
            ht_ref[0, r0:r1, :] = h2
            ct_ref[0, r0:r1, :] = c2


def _lstm_call(g, wihT, whhT, bg):
    return pl.pallas_call(
        _lstm_body,
        grid=(_T,),
        in_specs=[
            pl.BlockSpec((2, 1, _B, _EMBP), lambda t: (0, t, 0, 0)),
            pl.BlockSpec((4 * _HID, _EMBP), lambda t: (0, 0)),
            pl.BlockSpec((4 * _HID, _HID), lambda t: (0, 0)),
            pl.BlockSpec((1, 4 * _HID), lambda t: (0, 0)),
        ],
        out_specs=[
            pl.BlockSpec((1, _B, _HID), lambda t: (t, 0, 0)),
            pl.BlockSpec((1, _B, _HID), lambda t: (0, 0, 0)),
            pl.BlockSpec((1, _B, _HID), lambda t: (0, 0, 0)),
        ],
        out_shape=[
            jax.ShapeDtypeStruct((_T, _B, _HID), jnp.bfloat16),
            jax.ShapeDtypeStruct((1, _B, _HID), jnp.float32),
            jax.ShapeDtypeStruct((1, _B, _HID), jnp.float32),
        ],
        scratch_shapes=[
            pltpu.VMEM((_B, _HID), jnp.bfloat16),
            pltpu.VMEM((_B, _HID), jnp.float32),
        ],
        compiler_params=pltpu.CompilerParams(
            dimension_semantics=("arbitrary",),
            vmem_limit_bytes=100 * 1024 * 1024,
        ),
    )(g, wihT, whhT, bg)


def _dec_body(h_ref, wd_ref, bd_ref, wv_ref, bv_ref, out_ref, outv_ref):
    h = h_ref[0]
    nt = (((1,), (1,)), ((), ()))
    logits = lax.dot_general(wd_ref[...], h, nt,
                             preferred_element_type=jnp.float32) + bd_ref[...]
    sh = logits - jnp.max(logits, axis=0, keepdims=True)
    out_ref[0] = sh - jnp.log(jnp.sum(jnp.exp(sh), axis=0, keepdims=True))
    vlog = lax.dot_general(wv_ref[...], h, nt,
                           preferred_element_type=jnp.float32) + bv_ref[...]
    vsh = vlog - jnp.max(vlog, axis=0, keepdims=True)
    outv_ref[0] = vsh - jnp.log(jnp.sum(jnp.exp(vsh), axis=0, keepdims=True))


def _dec_call(hall, wd, bd, wv, bv):
    return pl.pallas_call(
        _dec_body,
        grid=(_T,),
        in_specs=[
            pl.BlockSpec((1, _B, _HID), lambda t: (t, 0, 0)),
            pl.BlockSpec((_VOCAB, _HID), lambda t: (0, 0)),
            pl.BlockSpec((_VOCAB, 1), lambda t: (0, 0)),
            pl.BlockSpec((_VVOCAB, _HID), lambda t: (0, 0)),
            pl.BlockSpec((_VVOCAB, 1), lambda t: (0, 0)),
        ],
        out_specs=[
            pl.BlockSpec((1, _VOCAB, _B), lambda t: (t, 0, 0)),
            pl.BlockSpec((1, _VVOCAB, _B), lambda t: (t, 0, 0)),
        ],
        out_shape=[
            jax.ShapeDtypeStruct((_T, _VOCAB, _B), jnp.float32),
            jax.ShapeDtypeStruct((_T, _VVOCAB, _B), jnp.float32),
        ],
        compiler_params=pltpu.CompilerParams(
            dimension_semantics=("arbitrary",),
            vmem_limit_bytes=100 * 1024 * 1024,
        ),
    )(hall, wd, bd, wv, bv)


def kernel(x1, x2, encoder, encoder_vel, W_ih, W_hh, b_ih, b_hh, W_dec, b_dec,
           W_dec_vel, b_dec_vel):
    info = plsc.get_sparse_core_info()
    nw = info.num_cores * info.num_subcores
    idx = jnp.concatenate([
        jnp.swapaxes(x1, 0, 1).reshape(-1),
        jnp.swapaxes(x2, 0, 1).reshape(-1),
    ]).astype(jnp.int32)
    idx3 = idx.reshape(nw, -1, _CHUNK)
    table_p = jnp.pad(encoder, ((0, 0), (0, _EMBP - _EMB)))
    rows = _sc_gather_rows(table_p, idx3)
    g = rows.reshape(2, _T, _B, _EMBP)

    bg = (b_ih + b_hh).reshape(1, 4 * _HID)
    bf16 = jnp.bfloat16
    wih_p = jnp.pad(W_ih, ((0, 0), (0, _EMBP - _EMB))).astype(bf16)
    hall, ht, ct = _lstm_call(g, wih_p, W_hh.astype(bf16), bg)

    outT, outvT = _dec_call(
        hall, W_dec.astype(bf16), b_dec.reshape(_VOCAB, 1),
        W_dec_vel.astype(bf16), b_dec_vel.reshape(_VVOCAB, 1))
    out = jnp.transpose(outT, (2, 0, 1))
    outv = jnp.transpose(outvT, (2, 0, 1))
    return (out, outv, (ht, ct))


# speedup vs baseline: 1.5825x; 1.0193x over previous
import functools

import jax
import jax.numpy as jnp
from jax import lax
from jax.experimental import pallas as pl
from jax.experimental.pallas import tpu as pltpu
from jax.experimental.pallas import tpu_sc as plsc

_VOCAB = 1000
_VVOCAB = 128
_EMB = 64
_HID = 512
_B = 1024
_T = 20
_TP = 24
_DB = 16
_CHUNK = 128
_EMBP = 128
_WAVE = 5


def _sc_gather_rows(table, idx):
    info = plsc.get_sparse_core_info()
    nc, ns = info.num_cores, info.num_subcores
    nw = nc * ns
    k = idx.shape[1]
    mesh = plsc.VectorSubcoreMesh(core_axis_name="c", subcore_axis_name="s")

    @functools.partial(
        pl.kernel,
        mesh=mesh,
        out_type=jax.ShapeDtypeStruct((nw, k, _CHUNK, _EMBP), jnp.float32),
        scratch_types=[
            pltpu.VMEM((k, _CHUNK), jnp.int32),
            pltpu.VMEM((_WAVE, _CHUNK, _EMBP), jnp.float32),
            pltpu.SemaphoreType.DMA,
        ],
    )
    def run(table_hbm, idx_hbm, out_hbm, idx_v, rows_v, sem):
        wid = lax.axis_index("s") * nc + lax.axis_index("c")
        pltpu.sync_copy(idx_hbm.at[wid], idx_v)
        for w in range(k // _WAVE):
            cps = [
                pltpu.async_copy(
                    table_hbm.at[idx_v.at[w * _WAVE + j]], rows_v.at[j], sem)
                for j in range(_WAVE)
            ]
            for cp in cps:
                cp.wait()
            pltpu.sync_copy(rows_v, out_hbm.at[wid].at[pl.ds(w * _WAVE, _WAVE)])

    return run(table, idx)


_NSPLIT = 2


def _sigmoid(x):
    return jnp.tanh(x * 0.5) * 0.5 + 0.5


def _fused_body(g_ref, wih_ref, whh_ref, bg_ref, wd_ref, bd_ref, wv_ref,
                bv_ref, out_ref, outv_ref, ht_ref, ct_ref, h_sc, c_sc):
    t = pl.program_id(0)
    nt = (((1,), (1,)), ((), ()))

    @pl.when(t == 0)
    def _():
        h_sc[...] = jnp.zeros_like(h_sc)
        c_sc[...] = jnp.zeros_like(c_sc)

    bs = _B // _NSPLIT
    for s in range(_NSPLIT):
        r0, r1 = s * bs, (s + 1) * bs
        x = (g_ref[0, 0, r0:r1, :] + g_ref[1, 0, r0:r1, :]).astype(jnp.bfloat16)
        h = h_sc[r0:r1, :]
        c = c_sc[r0:r1, :]
        gates = (lax.dot_general(x, wih_ref[...], nt,
                                 preferred_element_type=jnp.float32)
                 + lax.dot_general(h, whh_ref[...], nt,
                                   preferred_element_type=jnp.float32)
                 + bg_ref[...])
        gif = _sigmoid(gates[:, :2 * _HID])
        gg = jnp.tanh(gates[:, 2 * _HID:3 * _HID])
        go = _sigmoid(gates[:, 3 * _HID:])
        c2 = gif[:, _HID:] * c + gif[:, :_HID] * gg
        h2 = go * jnp.tanh(c2)
        h2b = h2.astype(jnp.bfloat16)
        h_sc[r0:r1, :] = h2b
        c_sc[r0:r1, :] = c2

        @pl.when(t == _T - 1)
        def _(h2=h2, c2=c2, r0=r0, r1=r1):
            ht_ref[0, r0:r1, :] = h2
            ct_ref[0, r0:r1, :] = c2

    hb = h_sc[...]
    logits = lax.dot_general(wd_ref[...], hb, nt,
                             preferred_element_type=jnp.float32) + bd_ref[...]
    sh = logits - jnp.max(logits, axis=0, keepdims=True)
    out_ref[0] = sh - jnp.log(jnp.sum(jnp.exp(sh), axis=0, keepdims=True))
    vlog = lax.dot_general(wv_ref[...], hb, nt,
                           preferred_element_type=jnp.float32) + bv_ref[...]
    vsh = vlog - jnp.max(vlog, axis=0, keepdims=True)
    outv_ref[0] = vsh - jnp.log(jnp.sum(jnp.exp(vsh), axis=0, keepdims=True))


def _fused_call(g, wih, whh, bg, wd, bd, wv, bv):
    return pl.pallas_call(
        _fused_body,
        grid=(_T,),
        in_specs=[
            pl.BlockSpec((2, 1, _B, _EMBP), lambda t: (0, t, 0, 0)),
            pl.BlockSpec((4 * _HID, _EMBP), lambda t: (0, 0)),
            pl.BlockSpec((4 * _HID, _HID), lambda t: (0, 0)),
            pl.BlockSpec((1, 4 * _HID), lambda t: (0, 0)),
            pl.BlockSpec((_VOCAB, _HID), lambda t: (0, 0)),
            pl.BlockSpec((_VOCAB, 1), lambda t: (0, 0)),
            pl.BlockSpec((_VVOCAB, _HID), lambda t: (0, 0)),
            pl.BlockSpec((_VVOCAB, 1), lambda t: (0, 0)),
        ],
        out_specs=[
            pl.BlockSpec((1, _VOCAB, _B), lambda t: (t, 0, 0)),
            pl.BlockSpec((1, _VVOCAB, _B), lambda t: (t, 0, 0)),
            pl.BlockSpec((1, _B, _HID), lambda t: (0, 0, 0)),
            pl.BlockSpec((1, _B, _HID), lambda t: (0, 0, 0)),
        ],
        out_shape=[
            jax.ShapeDtypeStruct((_T, _VOCAB, _B), jnp.float32),
            jax.ShapeDtypeStruct((_T, _VVOCAB, _B), jnp.float32),
            jax.ShapeDtypeStruct((1, _B, _HID), jnp.float32),
            jax.ShapeDtypeStruct((1, _B, _HID), jnp.float32),
        ],
        scratch_shapes=[
            pltpu.VMEM((_B, _HID), jnp.bfloat16),
            pltpu.VMEM((_B, _HID), jnp.float32),
        ],
        compiler_params=pltpu.CompilerParams(
            dimension_semantics=("arbitrary",),
            vmem_limit_bytes=100 * 1024 * 1024,
        ),
    )(g, wih, whh, bg, wd, bd, wv, bv)


def kernel(x1, x2, encoder, encoder_vel, W_ih, W_hh, b_ih, b_hh, W_dec, b_dec,
           W_dec_vel, b_dec_vel):
    info = plsc.get_sparse_core_info()
    nw = info.num_cores * info.num_subcores
    idx = jnp.concatenate([
        jnp.swapaxes(x1, 0, 1).reshape(-1),
        jnp.swapaxes(x2, 0, 1).reshape(-1),
    ]).astype(jnp.int32)
    idx3 = idx.reshape(nw, -1, _CHUNK)
    table_p = jnp.pad(encoder, ((0, 0), (0, _EMBP - _EMB)))
    rows = _sc_gather_rows(table_p, idx3)
    g = rows.reshape(2, _T, _B, _EMBP)

    bg = (b_ih + b_hh).reshape(1, 4 * _HID)
    bf16 = jnp.bfloat16
    wih_p = jnp.pad(W_ih, ((0, 0), (0, _EMBP - _EMB))).astype(bf16)
    outT, outvT, ht, ct = _fused_call(
        g, wih_p, W_hh.astype(bf16), bg,
        W_dec.astype(bf16), b_dec.reshape(_VOCAB, 1),
        W_dec_vel.astype(bf16), b_dec_vel.reshape(_VVOCAB, 1))
    out = jnp.transpose(outT, (2, 0, 1))
    outv = jnp.transpose(outvT, (2, 0, 1))
    return (out, outv, (ht, ct))
